# Initial kernel scaffold; baseline (speedup 1.0000x reference)
import jax
import jax.numpy as jnp
from jax.experimental import pallas as pl


def kernel(x, ref_points, Wv, bv, Woff, boff, Waw, baw, Wo, bo):
    raise NotImplementedError("write your pallas kernel here")


# speedup vs baseline: 23104.9978x; 23104.9978x over previous
import jax
import jax.numpy as jnp
from jax.experimental import pallas as pl

_DIM = 768
_P = 8
_N = 1024
_H = 32


def _dsaam_kernel(x_ref, rp_ref, Wv_ref, bv_ref, Wcat_ref, bcat_ref, Wo_ref,
                  bo_ref, out_ref):
    x = x_ref[0]

    value = jnp.dot(x, Wv_ref[...], preferred_element_type=jnp.float32)
    value = value + bv_ref[...]

    cat = jnp.dot(x, Wcat_ref[...], preferred_element_type=jnp.float32)
    cat = cat + bcat_ref[...]
    offx = cat[:, 0:_P]
    offy = cat[:, _P:2 * _P]
    awl = cat[:, 2 * _P:3 * _P]

    m = jnp.max(awl, axis=-1, keepdims=True)
    e = jnp.exp(awl - m)
    aw = e / jnp.sum(e, axis=-1, keepdims=True)

    rp = rp_ref[0]
    rx = rp[:, 0:1]
    ry = rp[:, 1:2]
    scale = (_H - 1) * 0.5
    gx = (jnp.clip(rx + offx, -1.0, 1.0) + 1.0) * scale
    gy = (jnp.clip(ry + offy, -1.0, 1.0) + 1.0) * scale

    mcol_i = jax.lax.broadcasted_iota(jnp.int32, (1, _N), 1)
    jcol = (mcol_i // _H).astype(jnp.float32)
    kcol = (mcol_i % _H).astype(jnp.float32)

    S = jnp.zeros((_N, _N), jnp.float32)
    for p in range(_P):
        wx = jnp.maximum(0.0, 1.0 - jnp.abs(gx[:, p:p + 1] - kcol))
        wy = jnp.maximum(0.0, 1.0 - jnp.abs(gy[:, p:p + 1] - jcol))
        S = S + (aw[:, p:p + 1] * wx) * wy

    sampled = jnp.dot(S, value, preferred_element_type=jnp.float32)
    out = jnp.dot(sampled, Wo_ref[...], preferred_element_type=jnp.float32)
    out_ref[0] = out + bo_ref[...]


def kernel(x, ref_points, Wv, bv, Woff, boff, Waw, baw, Wo, bo):
    B, N, C = x.shape
    Woff3 = Woff.reshape(C, _P, 2)
    Wcat = jnp.concatenate([Woff3[:, :, 0], Woff3[:, :, 1], Waw], axis=1)
    boff3 = boff.reshape(_P, 2)
    bcat = jnp.concatenate([boff3[:, 0], boff3[:, 1], baw]).reshape(1, 3 * _P)

    grid = (B,)
    out = pl.pallas_call(
        _dsaam_kernel,
        grid=grid,
        in_specs=[
            pl.BlockSpec((1, N, C), lambda b: (b, 0, 0)),
            pl.BlockSpec((1, N, 2), lambda b: (b, 0, 0)),
            pl.BlockSpec((C, C), lambda b: (0, 0)),
            pl.BlockSpec((1, C), lambda b: (0, 0)),
            pl.BlockSpec((C, 3 * _P), lambda b: (0, 0)),
            pl.BlockSpec((1, 3 * _P), lambda b: (0, 0)),
            pl.BlockSpec((C, C), lambda b: (0, 0)),
            pl.BlockSpec((1, C), lambda b: (0, 0)),
        ],
        out_specs=pl.BlockSpec((1, N, C), lambda b: (b, 0, 0)),
        out_shape=jax.ShapeDtypeStruct((B, N, C), jnp.float32),
    )(x, ref_points, Wv, bv.reshape(1, C), Wcat, bcat, Wo, bo.reshape(1, C))
    return out


# speedup vs baseline: 26786.2318x; 1.1593x over previous
import jax
import jax.numpy as jnp
from jax.experimental import pallas as pl

_DIM = 768
_P = 8
_N = 1024
_H = 32


def _dsaam_kernel(x_ref, rp_ref, Wv_ref, bv_ref, Wcat_ref, bcat_ref, Wo_ref,
                  bo_ref, out_ref):
    x = x_ref[0]

    value = jnp.dot(x, Wv_ref[...], preferred_element_type=jnp.float32)
    value = value + bv_ref[...]

    cat = jnp.dot(x, Wcat_ref[...], preferred_element_type=jnp.float32)
    cat = cat + bcat_ref[...]
    catT = cat.T
    offxT = catT[0:_P]
    offyT = catT[_P:2 * _P]
    awlT = catT[2 * _P:3 * _P]

    m = jnp.max(awlT, axis=0, keepdims=True)
    e = jnp.exp(awlT - m)
    awT = e / jnp.sum(e, axis=0, keepdims=True)

    rpT = rp_ref[0].T
    scale = (_H - 1) * 0.5
    gxT = (jnp.clip(rpT[0:1] + offxT, -1.0, 1.0) + 1.0) * scale
    gyT = (jnp.clip(rpT[1:2] + offyT, -1.0, 1.0) + 1.0) * scale

    mrow = jax.lax.broadcasted_iota(jnp.int32, (_N, 1), 0)
    jrow = (mrow // _H).astype(jnp.float32)
    krow = (mrow % _H).astype(jnp.float32)

    St = jnp.zeros((_N, _N), jnp.float32)
    for p in range(_P):
        wx = jnp.maximum(0.0, 1.0 - jnp.abs(gxT[p:p + 1] - krow))
        wy = jnp.maximum(0.0, 1.0 - jnp.abs(gyT[p:p + 1] - jrow))
        St = St + (awT[p:p + 1] * wx) * wy

    sampled = jax.lax.dot_general(St, value, (((0,), (0,)), ((), ())),
                                  preferred_element_type=jnp.float32)
    out = jnp.dot(sampled, Wo_ref[...], preferred_element_type=jnp.float32)
    out_ref[0] = out + bo_ref[...]


def kernel(x, ref_points, Wv, bv, Woff, boff, Waw, baw, Wo, bo):
    B, N, C = x.shape
    Woff3 = Woff.reshape(C, _P, 2)
    Wcat = jnp.concatenate([Woff3[:, :, 0], Woff3[:, :, 1], Waw], axis=1)
    boff3 = boff.reshape(_P, 2)
    bcat = jnp.concatenate([boff3[:, 0], boff3[:, 1], baw]).reshape(1, 3 * _P)

    grid = (B,)
    out = pl.pallas_call(
        _dsaam_kernel,
        grid=grid,
        in_specs=[
            pl.BlockSpec((1, N, C), lambda b: (b, 0, 0)),
            pl.BlockSpec((1, N, 2), lambda b: (b, 0, 0)),
            pl.BlockSpec((C, C), lambda b: (0, 0)),
            pl.BlockSpec((1, C), lambda b: (0, 0)),
            pl.BlockSpec((C, 3 * _P), lambda b: (0, 0)),
            pl.BlockSpec((1, 3 * _P), lambda b: (0, 0)),
            pl.BlockSpec((C, C), lambda b: (0, 0)),
            pl.BlockSpec((1, C), lambda b: (0, 0)),
        ],
        out_specs=pl.BlockSpec((1, N, C), lambda b: (b, 0, 0)),
        out_shape=jax.ShapeDtypeStruct((B, N, C), jnp.float32),
    )(x, ref_points, Wv, bv.reshape(1, C), Wcat, bcat, Wo, bo.reshape(1, C))
    return out


# speedup vs baseline: 48741.8812x; 1.8197x over previous
import jax
import jax.numpy as jnp
from jax.experimental import pallas as pl

_DIM = 768
_P = 8
_N = 1024
_H = 32


def _dsaam_kernel(x_ref, rp_ref, Wv_ref, bv_ref, Wcat_ref, bcat_ref, Wo_ref,
                  bo_ref, out_ref):
    x = x_ref[0]

    value = jnp.dot(x, Wv_ref[...], preferred_element_type=jnp.float32)
    value = value + bv_ref[...]

    cat = jnp.dot(x, Wcat_ref[...], preferred_element_type=jnp.float32)
    cat = cat + bcat_ref[...]
    catT = cat.T
    offxT = catT[0:_P]
    offyT = catT[_P:2 * _P]
    awlT = catT[2 * _P:3 * _P]

    m = jnp.max(awlT, axis=0, keepdims=True)
    e = jnp.exp(awlT - m)
    awT = e / jnp.sum(e, axis=0, keepdims=True)

    rpT = rp_ref[0].T
    scale = (_H - 1) * 0.5
    gxT = (jnp.clip(rpT[0:1] + offxT, -1.0, 1.0) + 1.0) * scale
    gyT = (jnp.clip(rpT[1:2] + offyT, -1.0, 1.0) + 1.0) * scale

    grow = jax.lax.broadcasted_iota(jnp.int32, (_H, 1), 0).astype(jnp.float32)

    St3 = jnp.zeros((_H, _H, _N), jnp.float32)
    for p in range(_P):
        Xp = jnp.maximum(0.0, 1.0 - jnp.abs(gxT[p:p + 1] - grow))
        Yp = jnp.maximum(0.0, 1.0 - jnp.abs(gyT[p:p + 1] - grow))
        Yp = awT[p:p + 1] * Yp
        St3 = St3 + Yp[:, None, :] * Xp[None, :, :]
    St = St3.reshape(_N, _N)

    sampled = jax.lax.dot_general(St, value, (((0,), (0,)), ((), ())),
                                  preferred_element_type=jnp.float32)
    out = jnp.dot(sampled, Wo_ref[...], preferred_element_type=jnp.float32)
    out_ref[0] = out + bo_ref[...]


def kernel(x, ref_points, Wv, bv, Woff, boff, Waw, baw, Wo, bo):
    B, N, C = x.shape
    Woff3 = Woff.reshape(C, _P, 2)
    Wcat = jnp.concatenate([Woff3[:, :, 0], Woff3[:, :, 1], Waw], axis=1)
    boff3 = boff.reshape(_P, 2)
    bcat = jnp.concatenate([boff3[:, 0], boff3[:, 1], baw]).reshape(1, 3 * _P)

    grid = (B,)
    out = pl.pallas_call(
        _dsaam_kernel,
        grid=grid,
        in_specs=[
            pl.BlockSpec((1, N, C), lambda b: (b, 0, 0)),
            pl.BlockSpec((1, N, 2), lambda b: (b, 0, 0)),
            pl.BlockSpec((C, C), lambda b: (0, 0)),
            pl.BlockSpec((1, C), lambda b: (0, 0)),
            pl.BlockSpec((C, 3 * _P), lambda b: (0, 0)),
            pl.BlockSpec((1, 3 * _P), lambda b: (0, 0)),
            pl.BlockSpec((C, C), lambda b: (0, 0)),
            pl.BlockSpec((1, C), lambda b: (0, 0)),
        ],
        out_specs=pl.BlockSpec((1, N, C), lambda b: (b, 0, 0)),
        out_shape=jax.ShapeDtypeStruct((B, N, C), jnp.float32),
    )(x, ref_points, Wv, bv.reshape(1, C), Wcat, bcat, Wo, bo.reshape(1, C))
    return out


# speedup vs baseline: 54078.5777x; 1.1095x over previous
import jax
import jax.numpy as jnp
from jax.experimental import pallas as pl

_DIM = 768
_P = 8
_N = 1024
_H = 32


def _dsaam_kernel(x_ref, rp_ref, Wv_ref, bv_ref, Wcat_ref, bcat_ref, Wo_ref,
                  bo_ref, out_ref):
    x = x_ref[0]

    value = jnp.dot(x, Wv_ref[...], preferred_element_type=jnp.float32)
    value = value + bv_ref[...]

    cat = jnp.dot(x, Wcat_ref[...], preferred_element_type=jnp.float32)
    cat = cat + bcat_ref[...]
    catT = cat.T
    offxT = catT[0:_P]
    offyT = catT[_P:2 * _P]
    awlT = catT[2 * _P:3 * _P]

    m = jnp.max(awlT, axis=0, keepdims=True)
    e = jnp.exp(awlT - m)
    awT = e / jnp.sum(e, axis=0, keepdims=True)

    rpT = rp_ref[0].T
    scale = (_H - 1) * 0.5
    gxT = (jnp.clip(rpT[0:1] + offxT, -1.0, 1.0) + 1.0) * scale
    gyT = (jnp.clip(rpT[1:2] + offyT, -1.0, 1.0) + 1.0) * scale

    grow = jax.lax.broadcasted_iota(jnp.int32, (_H, 1), 0).astype(jnp.float32)

    St3 = jnp.zeros((_H, _H, _N), jnp.bfloat16)
    for p in range(_P):
        Xp = jnp.maximum(0.0, 1.0 - jnp.abs(gxT[p:p + 1] - grow))
        Yp = jnp.maximum(0.0, 1.0 - jnp.abs(gyT[p:p + 1] - grow))
        Yp = awT[p:p + 1] * Yp
        St3 = St3 + Yp.astype(jnp.bfloat16)[:, None, :] * \
            Xp.astype(jnp.bfloat16)[None, :, :]
    St = St3.reshape(_N, _N)

    sampled = jax.lax.dot_general(St, value.astype(jnp.bfloat16),
                                  (((0,), (0,)), ((), ())),
                                  preferred_element_type=jnp.float32)
    out = jnp.dot(sampled, Wo_ref[...], preferred_element_type=jnp.float32)
    out_ref[0] = out + bo_ref[...]


def kernel(x, ref_points, Wv, bv, Woff, boff, Waw, baw, Wo, bo):
    B, N, C = x.shape
    Woff3 = Woff.reshape(C, _P, 2)
    Wcat = jnp.concatenate([Woff3[:, :, 0], Woff3[:, :, 1], Waw], axis=1)
    boff3 = boff.reshape(_P, 2)
    bcat = jnp.concatenate([boff3[:, 0], boff3[:, 1], baw]).reshape(1, 3 * _P)

    grid = (B,)
    out = pl.pallas_call(
        _dsaam_kernel,
        grid=grid,
        in_specs=[
            pl.BlockSpec((1, N, C), lambda b: (b, 0, 0)),
            pl.BlockSpec((1, N, 2), lambda b: (b, 0, 0)),
            pl.BlockSpec((C, C), lambda b: (0, 0)),
            pl.BlockSpec((1, C), lambda b: (0, 0)),
            pl.BlockSpec((C, 3 * _P), lambda b: (0, 0)),
            pl.BlockSpec((1, 3 * _P), lambda b: (0, 0)),
            pl.BlockSpec((C, C), lambda b: (0, 0)),
            pl.BlockSpec((1, C), lambda b: (0, 0)),
        ],
        out_specs=pl.BlockSpec((1, N, C), lambda b: (b, 0, 0)),
        out_shape=jax.ShapeDtypeStruct((B, N, C), jnp.float32),
    )(x, ref_points, Wv, bv.reshape(1, C), Wcat, bcat, Wo, bo.reshape(1, C))
    return out
